# Initial kernel scaffold; baseline (speedup 1.0000x reference)
import jax
import jax.numpy as jnp
from jax.experimental import pallas as pl


def kernel(var_node_features, con_node_features, edge_index_var, edge_features_var, rhs, edge_index_con, edge_features_con, asums, params):
    raise NotImplementedError("write your pallas kernel here")


# speedup vs baseline: 12.2703x; 12.2703x over previous
import functools

import jax
import jax.numpy as jnp
from jax import lax
from jax.experimental import pallas as pl
from jax.experimental.pallas import tpu as pltpu
from jax.experimental.pallas import tpu_sc as plsc

N = 25000
NE = 800000
D = 32
DW = 16
KM = 128
NW = 32
ROWS = 6400
RW = ROWS // NW
PADN = ROWS * KM - NE
SENT = N
NPAD = N + 8
BB = 1000
GB = N // BB


@functools.lru_cache(maxsize=None)
def _build_spmm():
    mesh = plsc.VectorSubcoreMesh(core_axis_name="c", subcore_axis_name="s")
    return functools.partial(
        pl.kernel,
        out_type=jax.ShapeDtypeStruct((2, NPAD, D), jnp.float32),
        mesh=mesh,
        scratch_types=[
            pltpu.VMEM((RW, KM), jnp.int32),
            pltpu.VMEM((RW, KM), jnp.int32),
            pltpu.VMEM((KM, D), jnp.float32),
            pltpu.VMEM_SHARED((NPAD, D), jnp.float32),
            pltpu.SemaphoreType.DMA,
        ],
        compiler_params=pltpu.CompilerParams(use_tc_tiling_on_sc=False),
    )(_spmm_body)


def _spmm(m, srcg, dstg, zero):
    return _build_spmm()(m, srcg, dstg, zero)


def _spmm_body(m_hbm, srcg_hbm, dstg_hbm, zero_hbm, out_hbm, idx_v, didx_v, rows_v, acc, sem):
    c = lax.axis_index("c")
    s = lax.axis_index("s")
    wid = s * 2 + c
    base = wid * RW
    pltpu.sync_copy(srcg_hbm.at[pl.ds(base, RW)], idx_v)
    pltpu.sync_copy(dstg_hbm.at[pl.ds(base, RW)], didx_v)

    @pl.when(s == 0)
    def _zero():
        pltpu.sync_copy(zero_hbm, acc)

    plsc.subcore_barrier()

    def body(i, carry):
        pltpu.async_copy(m_hbm.at[idx_v.at[i]], rows_v, sem).wait()
        pltpu.sync_copy(rows_v, acc.at[didx_v.at[i]], add=True)
        return carry

    lax.fori_loop(0, RW, body, 0)
    plsc.subcore_barrier()

    @pl.when(s == 0)
    def _writeback():
        pltpu.sync_copy(acc, out_hbm.at[c])


@functools.lru_cache(maxsize=None)
def _build_deg():
    mesh = plsc.VectorSubcoreMesh(core_axis_name="c", subcore_axis_name="s")
    return functools.partial(
        pl.kernel,
        out_type=jax.ShapeDtypeStruct((2, NPAD, DW), jnp.float32),
        mesh=mesh,
        scratch_types=[
            pltpu.VMEM((RW, KM), jnp.int32),
            pltpu.VMEM((KM, DW), jnp.float32),
            pltpu.VMEM_SHARED((NPAD, DW), jnp.float32),
        ],
        compiler_params=pltpu.CompilerParams(use_tc_tiling_on_sc=False),
    )(_deg_body)


def _deg(srcd, ones_w, zero_w):
    return _build_deg()(srcd, ones_w, zero_w)


def _deg_body(srcd_hbm, ones_hbm, zero_hbm, out_hbm, idx_v, ones_v, acc):
    c = lax.axis_index("c")
    s = lax.axis_index("s")
    wid = s * 2 + c
    base = wid * RW
    pltpu.sync_copy(srcd_hbm.at[pl.ds(base, RW)], idx_v)
    pltpu.sync_copy(ones_hbm, ones_v)

    @pl.when(s == 0)
    def _zero():
        pltpu.sync_copy(zero_hbm, acc)

    plsc.subcore_barrier()

    def body(i, carry):
        pltpu.sync_copy(ones_v, acc.at[idx_v.at[i]], add=True)
        return carry

    lax.fori_loop(0, RW, body, 0)
    plsc.subcore_barrier()

    @pl.when(s == 0)
    def _writeback():
        pltpu.sync_copy(acc, out_hbm.at[c])


def _full(shape):
    return pl.BlockSpec(shape, lambda i: tuple(0 for _ in shape))


def _rows(cols, b=BB):
    return pl.BlockSpec((b, cols), lambda i: (i, 0))


_AGG_SPEC = pl.BlockSpec((2, BB, D), lambda i: (0, i, 0))
_DEG_SPEC = pl.BlockSpec((2, BB, DW), lambda i: (0, i, 0))


def _col_is_last(shape):
    return lax.broadcasted_iota(jnp.int32, shape, 1) == (D - 1)


def _mlp2_body(x_ref, w1_ref, b1_ref, w2_ref, b2_ref, o_ref):
    h = jnp.maximum(x_ref[...] @ w1_ref[...] + b1_ref[...], 0.0)
    o_ref[...] = h @ w2_ref[...] + b2_ref[...]


def _mlp2_tc(x, w1, b1, w2, b2):
    cin = x.shape[1]
    return pl.pallas_call(
        _mlp2_body,
        grid=(GB,),
        in_specs=[_rows(cin), _full(w1.shape), _full(b1.shape),
                  _full(w2.shape), _full(b2.shape)],
        out_specs=_rows(D),
        out_shape=jax.ShapeDtypeStruct((N, D), jnp.float32),
    )(x, w1, b1, w2, b2)


def _v2c_msg_body(x_ref, ef_ref, dg_ref, w1h, b1h, w2h, b2h, w1m, b1m, w2m, b2m, o_ref):
    x = x_ref[...]
    ef = ef_ref[...]
    deg = (dg_ref[0] + dg_ref[1])[:, 0:1]
    norm = jnp.where(deg > 0.5, 1.0 / deg, 0.0)
    t = jax.nn.sigmoid(x @ w1h[...] + b1h[...])
    va = (t @ w2h[...] + b2h[...])[:, 0:1] * ef
    u = jnp.maximum((ef * x) @ w1m[...] + b1m[...], 0.0)
    out = (u @ w2m[...] + b2m[...]) * norm
    o_ref[...] = jnp.where(_col_is_last(out.shape), va, out)


def _v2c_msg(x, ef, dg, wh, wm):
    return pl.pallas_call(
        _v2c_msg_body,
        grid=(GB,),
        in_specs=[_rows(D), _rows(1), _DEG_SPEC] +
                 [_full(w.shape) for w in wh] + [_full(w.shape) for w in wm],
        out_specs=_rows(D),
        out_shape=jax.ShapeDtypeStruct((N, D), jnp.float32),
    )(x, ef, dg, *wh, *wm)


def _v2c_upd_body(ag_ref, old_ref, rhs_ref, wr, br, o_ref):
    aggr = ag_ref[0] + ag_ref[1]
    main = jnp.maximum(aggr + old_ref[...] @ wr[...] + br[...], 0.0)
    last = aggr[:, D - 1:D] - rhs_ref[...]
    o_ref[...] = jnp.where(_col_is_last(main.shape), last, main)


def _v2c_upd(ag, old, rhs2, wr, br):
    return pl.pallas_call(
        _v2c_upd_body,
        grid=(GB,),
        in_specs=[_AGG_SPEC, _rows(D), _rows(1), _full(wr.shape), _full(br.shape)],
        out_specs=_rows(D),
        out_shape=jax.ShapeDtypeStruct((N, D), jnp.float32),
    )(ag, old, rhs2, wr, br)


def _c2v_msg_body(x_ref, ef_ref, dg_ref, w1m, b1m, w2m, b2m, o_ref):
    x = x_ref[...]
    ef = ef_ref[...]
    deg = (dg_ref[0] + dg_ref[1])[:, 0:1]
    norm = jnp.where(deg > 0.5, 1.0 / deg, 0.0)
    u = jnp.maximum((ef * x) @ w1m[...] + b1m[...], 0.0)
    out = u @ w2m[...] + b2m[...]
    bscal = x[:, D - 1:D] * ef
    o_ref[...] = norm * jnp.where(_col_is_last(out.shape), bscal, out)


def _c2v_msg(x, ef, dg, wm):
    return pl.pallas_call(
        _c2v_msg_body,
        grid=(GB,),
        in_specs=[_rows(D), _rows(1), _DEG_SPEC] + [_full(w.shape) for w in wm],
        out_specs=_rows(D),
        out_shape=jax.ShapeDtypeStruct((N, D), jnp.float32),
    )(x, ef, dg, *wm)


def _c2v_upd_body(ag_ref, xd_ref, w1h, b1h, w2h, b2h, wr, br, o_ref):
    aggr = ag_ref[0] + ag_ref[1]
    xd = xd_ref[...]
    t = jax.nn.sigmoid(xd @ w1h[...] + b1h[...])
    a = (t @ w2h[...] + b2h[...])[:, 0:1]
    main = jnp.maximum(aggr + xd @ wr[...] + br[...], 0.0)
    last = a * aggr[:, D - 1:D]
    o_ref[...] = jnp.where(_col_is_last(main.shape), last, main)


def _c2v_upd(ag, xd, wh, wr, br):
    return pl.pallas_call(
        _c2v_upd_body,
        grid=(GB,),
        in_specs=[_AGG_SPEC, _rows(D)] + [_full(w.shape) for w in wh] +
                 [_full(wr.shape), _full(br.shape)],
        out_specs=_rows(D),
        out_shape=jax.ShapeDtypeStruct((N, D), jnp.float32),
    )(ag, xd, *wh, wr, br)


def _head_body(x_ref, *refs):
    o_ref = refs[-1]
    ws = refs[:-1]
    x = x_ref[...]
    for i in range(5):
        x = jnp.maximum(x @ ws[2 * i][...] + ws[2 * i + 1][...], 0.0)
    lg = x @ ws[10][...] + ws[11][...]
    o_ref[...] = jax.nn.log_softmax(lg, axis=1)


def _head(x, ws):
    return pl.pallas_call(
        _head_body,
        grid=(GB,),
        in_specs=[_rows(D)] + [_full(w.shape) for w in ws],
        out_specs=_rows(2),
        out_shape=jax.ShapeDtypeStruct((N, 2), jnp.float32),
    )(x, *ws)


def _pad_mlp2(p, pre):
    w1 = jnp.pad(p[pre + '_W1'], ((0, 0), (0, 1)))
    b1 = jnp.pad(p[pre + '_b1'], (0, 1)).reshape(1, D)
    w2 = jnp.pad(p[pre + '_W2'], ((0, 1), (0, 1)))
    b2 = jnp.pad(p[pre + '_b2'], (0, 1)).reshape(1, D)
    return (w1, b1, w2, b2)


def _pad_h2v(p, l):
    w1 = jnp.pad(p['h2v%d_W1' % l], ((0, 0), (0, 1)))
    b1 = jnp.pad(p['h2v%d_b1' % l], (0, 1)).reshape(1, D)
    w2 = jnp.pad(p['h2v%d_W2' % l], ((0, 1), (0, D - 1)))
    b2 = jnp.pad(p['h2v%d_b2' % l], (0, D - 1)).reshape(1, D)
    return (w1, b1, w2, b2)


def _pad_root(p, pre):
    wr = jnp.pad(p[pre + '_root'], ((0, 0), (0, 1)))
    br = jnp.pad(p[pre + '_bias'], (0, 1)).reshape(1, D)
    return wr, br


def _edge_layouts(src, dst):
    pad0 = jnp.zeros((PADN,), jnp.int32)
    pads = jnp.full((PADN,), SENT, jnp.int32)
    g_src = jnp.concatenate([src, pad0]).reshape(ROWS, KM)
    g_dst = jnp.concatenate([dst, pads]).reshape(ROWS, KM)
    d_src = jnp.concatenate([src, pads]).reshape(ROWS, KM)
    return g_src, g_dst, d_src


def _impl(vnf, cnf, eiv, efv, rhs, eic, efc, asums, params):
    del asums
    p = params
    sv = eiv[0].astype(jnp.int32)
    dv = eiv[1].astype(jnp.int32)
    sc = eic[0].astype(jnp.int32)
    dc = eic[1].astype(jnp.int32)
    svg, dvg, svd = _edge_layouts(sv, dv)
    scg, dcg, scd = _edge_layouts(sc, dc)
    zero_d = jnp.zeros((NPAD, D), jnp.float32)
    zero_w = jnp.zeros((NPAD, DW), jnp.float32)
    ones_w = jnp.ones((KM, DW), jnp.float32)
    efv = efv.astype(jnp.float32)
    efc = efc.astype(jnp.float32)
    rhs2 = rhs.astype(jnp.float32).reshape(N, 1)

    deg_v = _deg(svd, ones_w, zero_w)
    deg_c = _deg(scd, ones_w, zero_w)

    v0 = _mlp2_tc(vnf, *_pad_mlp2(p, 'con_mlp'))
    c0 = _mlp2_tc(cnf, *_pad_mlp2(p, 'var_mlp'))

    x_src, old_cons, vars_ = v0, c0, v0
    for l in (1, 2, 3, 4):
        wh = _pad_h2v(p, l)
        wm_v = _pad_mlp2(p, 'v2c%d' % l)
        mv = _v2c_msg(x_src, efv, deg_v, wh, wm_v)
        ag = _spmm(mv, svg, dvg, zero_d)
        cons = _v2c_upd(ag, old_cons, rhs2, *_pad_root(p, 'v2c%d' % l))

        wm_c = _pad_mlp2(p, 'c2v%d' % l)
        mc = _c2v_msg(cons, efc, deg_c, wm_c)
        ag2 = _spmm(mc, scg, dcg, zero_d)
        wr, br = _pad_root(p, 'c2v%d' % l)
        vars_ = _c2v_upd(ag2, vars_, wh, wr, br)
        x_src, old_cons = vars_, cons

    ws = []
    for i in range(1, 6):
        ws += [p['fc%d_W' % i], p['fc%d_b' % i].reshape(1, D)]
    ws += [p['fc6_W'], p['fc6_b'].reshape(1, 2)]
    return _head(vars_, ws)


def kernel(var_node_features, con_node_features, edge_index_var, edge_features_var, rhs, edge_index_con, edge_features_con, asums, params):
    return _impl(var_node_features, con_node_features, edge_index_var,
                 edge_features_var, rhs, edge_index_con, edge_features_con,
                 asums, params)
